# baseline (speedup 1.0000x reference)
import functools

import jax
import jax.numpy as jnp
from jax import lax
from jax.experimental import pallas as pl
from jax.experimental.pallas import tpu as pltpu
from jax.experimental.pallas import tpu_sc as plsc

_BATCH = 16384
_DIM = 32
_NUM_WORKERS = 32
_BPW = _BATCH // _NUM_WORKERS
_L = 16
_W = 128
_NSLOT = 4
_NGRP = _BPW // _L


def _gmf_body(users_hbm, items_hbm, ut_hbm, it_hbm, out_hbm,
              idx_u, idx_i, ring_u, ring_i, buf, *sems):
    sem_u = sems[:_NSLOT]
    sem_i = sems[_NSLOT:]
    wid = lax.axis_index("s") * 2 + lax.axis_index("c")
    base = wid * _BPW
    pltpu.sync_copy(users_hbm.at[pl.ds(base, _BPW)], idx_u)
    pltpu.sync_copy(items_hbm.at[pl.ds(base, _BPW)], idx_i)

    lane = lax.iota(jnp.int32, _L)
    dummy = out_hbm.at[pl.ds(0, _DIM), pl.ds(0, _W)]

    def issue(ou_vec, oi_vec, l):
        slot = l % _NSLOT
        ou = pl.multiple_of(ou_vec[l], _W)
        oi = pl.multiple_of(oi_vec[l], _W)
        for r0 in (0, _DIM // 2):
            sl_r = pl.ds(r0, _DIM // 2)
            pltpu.async_copy(ut_hbm.at[sl_r, pl.ds(ou, _W)],
                             ring_u.at[slot, sl_r], sem_u[slot])
            pltpu.async_copy(it_hbm.at[sl_r, pl.ds(oi, _W)],
                             ring_i.at[slot, sl_r], sem_i[slot])

    def process(ru_vec, ri_vec, l, row):
        slot = l % _NSLOT
        pltpu.make_async_copy(dummy, ring_u.at[slot], sem_u[slot]).wait()
        pltpu.make_async_copy(dummy, ring_i.at[slot], sem_i[slot]).wait()
        slot_splat = jnp.full((_L,), slot, jnp.int32)
        cu = jnp.broadcast_to(ru_vec[l], (_L,))
        ci = jnp.broadcast_to(ri_vec[l], (_L,))
        for h in (0, _L):
            u = plsc.load_gather(ring_u, [slot_splat, lane + h, cu])
            v = plsc.load_gather(ring_i, [slot_splat, lane + h, ci])
            buf[row, pl.ds(h, _L)] = u * v

    def body(g, carry):
        gp = jnp.maximum(g - 1, 0)
        vu = idx_u[pl.ds(g * _L, _L)]
        vi = idx_i[pl.ds(g * _L, _L)]
        pu = idx_u[pl.ds(gp * _L, _L)]
        pi = idx_i[pl.ds(gp * _L, _L)]
        ou_vec = (vu >> 7) << 7
        oi_vec = (vi >> 7) << 7
        ru_vec = vu & (_W - 1)
        ri_vec = vi & (_W - 1)
        rpu_vec = pu & (_W - 1)
        rpi_vec = pi & (_W - 1)

        for l in range(_NSLOT):
            @pl.when(g > 0)
            def _(l=l):
                process(rpu_vec, rpi_vec, _L - _NSLOT + l,
                        gp * _L + _L - _NSLOT + l)
            issue(ou_vec, oi_vec, l)
        for l in range(_NSLOT, _L):
            process(ru_vec, ri_vec, l - _NSLOT, g * _L + l - _NSLOT)
            issue(ou_vec, oi_vec, l)
        return carry

    lax.fori_loop(0, _NGRP, body, 0)

    g_last = _NGRP - 1
    vu = idx_u[pl.ds(g_last * _L, _L)] & (_W - 1)
    vi = idx_i[pl.ds(g_last * _L, _L)] & (_W - 1)
    for l in range(_NSLOT):
        process(vu, vi, _L - _NSLOT + l, g_last * _L + _L - _NSLOT + l)

    pltpu.sync_copy(buf, out_hbm.at[pl.ds(base, _BPW)])


@jax.jit
def kernel(users, items, user_table, item_table):
    mesh = plsc.VectorSubcoreMesh(core_axis_name="c", subcore_axis_name="s")
    run = functools.partial(
        pl.kernel,
        mesh=mesh,
        compiler_params=pltpu.CompilerParams(needs_layout_passes=False),
        out_type=jax.ShapeDtypeStruct((_BATCH, _DIM), jnp.float32),
        scratch_types=[
            pltpu.VMEM((_BPW,), jnp.int32),
            pltpu.VMEM((_BPW,), jnp.int32),
            pltpu.VMEM((_NSLOT, _DIM, _W), jnp.float32),
            pltpu.VMEM((_NSLOT, _DIM, _W), jnp.float32),
            pltpu.VMEM((_BPW, _DIM), jnp.float32),
        ] + [pltpu.SemaphoreType.DMA] * (2 * _NSLOT),
    )(_gmf_body)
    return run(users.astype(jnp.int32), items.astype(jnp.int32),
               user_table.T, item_table.T)


# speedup vs baseline: 1.0050x; 1.0050x over previous
import functools

import jax
import jax.numpy as jnp
from jax import lax
from jax.experimental import pallas as pl
from jax.experimental.pallas import tpu as pltpu
from jax.experimental.pallas import tpu_sc as plsc

_BATCH = 16384
_DIM = 32
_NUM_WORKERS = 32
_BPW = _BATCH // _NUM_WORKERS
_L = 16
_W = 128
_NSLOT = 4
_NGRP = _BPW // _L


def _gmf_body(users_hbm, items_hbm, ut_hbm, it_hbm, out_hbm,
              idx_u, idx_i, ring_u, ring_i, buf, *sems):
    sem_u = sems[:_NSLOT]
    sem_i = sems[_NSLOT:]
    wid = lax.axis_index("s") * 2 + lax.axis_index("c")
    base = wid * _BPW
    pltpu.sync_copy(users_hbm.at[pl.ds(base, _BPW)], idx_u)
    pltpu.sync_copy(items_hbm.at[pl.ds(base, _BPW)], idx_i)

    lane = lax.iota(jnp.int32, _L)
    dummy = out_hbm.at[pl.ds(0, _DIM), pl.ds(0, _W)]

    def issue(ou_vec, oi_vec, l):
        slot = l % _NSLOT
        ou = pl.multiple_of(ou_vec[l], _W)
        oi = pl.multiple_of(oi_vec[l], _W)
        pltpu.async_copy(ut_hbm.at[pl.ds(0, _DIM), pl.ds(ou, _W)],
                         ring_u.at[slot], sem_u[slot])
        pltpu.async_copy(it_hbm.at[pl.ds(0, _DIM), pl.ds(oi, _W)],
                         ring_i.at[slot], sem_i[slot])

    def process(ru_vec, ri_vec, l, row):
        slot = l % _NSLOT
        pltpu.make_async_copy(dummy, ring_u.at[slot], sem_u[slot]).wait()
        pltpu.make_async_copy(dummy, ring_i.at[slot], sem_i[slot]).wait()
        slot_splat = jnp.full((_L,), slot, jnp.int32)
        cu = jnp.broadcast_to(ru_vec[l], (_L,))
        ci = jnp.broadcast_to(ri_vec[l], (_L,))
        for h in (0, _L):
            u = plsc.load_gather(ring_u, [slot_splat, lane + h, cu])
            v = plsc.load_gather(ring_i, [slot_splat, lane + h, ci])
            buf[row, pl.ds(h, _L)] = u * v

    def body(g, carry):
        gp = jnp.maximum(g - 1, 0)
        vu = idx_u[pl.ds(g * _L, _L)]
        vi = idx_i[pl.ds(g * _L, _L)]
        pu = idx_u[pl.ds(gp * _L, _L)]
        pi = idx_i[pl.ds(gp * _L, _L)]
        ou_vec = (vu >> 7) << 7
        oi_vec = (vi >> 7) << 7
        ru_vec = vu & (_W - 1)
        ri_vec = vi & (_W - 1)
        rpu_vec = pu & (_W - 1)
        rpi_vec = pi & (_W - 1)

        for l in range(_NSLOT):
            @pl.when(g > 0)
            def _(l=l):
                process(rpu_vec, rpi_vec, _L - _NSLOT + l,
                        gp * _L + _L - _NSLOT + l)
            issue(ou_vec, oi_vec, l)
        for l in range(_NSLOT, _L):
            process(ru_vec, ri_vec, l - _NSLOT, g * _L + l - _NSLOT)
            issue(ou_vec, oi_vec, l)
        return carry

    lax.fori_loop(0, _NGRP, body, 0)

    g_last = _NGRP - 1
    vu = idx_u[pl.ds(g_last * _L, _L)] & (_W - 1)
    vi = idx_i[pl.ds(g_last * _L, _L)] & (_W - 1)
    for l in range(_NSLOT):
        process(vu, vi, _L - _NSLOT + l, g_last * _L + _L - _NSLOT + l)

    pltpu.sync_copy(buf, out_hbm.at[pl.ds(base, _BPW)])


@jax.jit
def kernel(users, items, user_table, item_table):
    mesh = plsc.VectorSubcoreMesh(core_axis_name="c", subcore_axis_name="s")
    run = functools.partial(
        pl.kernel,
        mesh=mesh,
        compiler_params=pltpu.CompilerParams(needs_layout_passes=False),
        out_type=jax.ShapeDtypeStruct((_BATCH, _DIM), jnp.float32),
        scratch_types=[
            pltpu.VMEM((_BPW,), jnp.int32),
            pltpu.VMEM((_BPW,), jnp.int32),
            pltpu.VMEM((_NSLOT, _DIM, _W), jnp.float32),
            pltpu.VMEM((_NSLOT, _DIM, _W), jnp.float32),
            pltpu.VMEM((_BPW, _DIM), jnp.float32),
        ] + [pltpu.SemaphoreType.DMA] * (2 * _NSLOT),
    )(_gmf_body)
    return run(users.astype(jnp.int32), items.astype(jnp.int32),
               user_table.T, item_table.T)


# speedup vs baseline: 1.0146x; 1.0095x over previous
import functools

import jax
import jax.numpy as jnp
from jax import lax
from jax.experimental import pallas as pl
from jax.experimental.pallas import tpu as pltpu
from jax.experimental.pallas import tpu_sc as plsc

_BATCH = 16384
_DIM = 32
_NUM_WORKERS = 32
_BPW = _BATCH // _NUM_WORKERS
_L = 16
_W = 128
_NSLOT = 8
_NGRP = _BPW // _L
_HALF = _BPW // 2


def _gmf_body(users_hbm, items_hbm, ut_hbm, it_hbm, out_hbm,
              idx_u, idx_i, ring_u, ring_i, buf, *sems):
    sem_u = sems[:_NSLOT]
    sem_i = sems[_NSLOT:]
    wid = lax.axis_index("s") * 2 + lax.axis_index("c")
    base = wid * _BPW
    pltpu.sync_copy(users_hbm.at[pl.ds(base, _BPW)], idx_u)
    pltpu.sync_copy(items_hbm.at[pl.ds(base, _BPW)], idx_i)

    lane = lax.iota(jnp.int32, _L)
    dummy = out_hbm.at[pl.ds(0, _DIM), pl.ds(0, _W)]

    def issue(ou_vec, oi_vec, l):
        slot = l % _NSLOT
        ou = pl.multiple_of(ou_vec[l], _W)
        oi = pl.multiple_of(oi_vec[l], _W)
        pltpu.async_copy(ut_hbm.at[pl.ds(0, _DIM), pl.ds(ou, _W)],
                         ring_u.at[slot], sem_u[slot])
        pltpu.async_copy(it_hbm.at[pl.ds(0, _DIM), pl.ds(oi, _W)],
                         ring_i.at[slot], sem_i[slot])

    def process(ru_vec, ri_vec, l, row):
        slot = l % _NSLOT
        pltpu.make_async_copy(dummy, ring_u.at[slot], sem_u[slot]).wait()
        pltpu.make_async_copy(dummy, ring_i.at[slot], sem_i[slot]).wait()
        slot_splat = jnp.full((_L,), slot, jnp.int32)
        cu = jnp.broadcast_to(ru_vec[l], (_L,))
        ci = jnp.broadcast_to(ri_vec[l], (_L,))
        for h in (0, _L):
            u = plsc.load_gather(ring_u, [slot_splat, lane + h, cu])
            v = plsc.load_gather(ring_i, [slot_splat, lane + h, ci])
            buf[row & (_HALF - 1), pl.ds(h, _L)] = u * v

    def body(g, carry):
        gp = jnp.maximum(g - 1, 0)
        vu = idx_u[pl.ds(g * _L, _L)]
        vi = idx_i[pl.ds(g * _L, _L)]
        pu = idx_u[pl.ds(gp * _L, _L)]
        pi = idx_i[pl.ds(gp * _L, _L)]
        ou_vec = (vu >> 7) << 7
        oi_vec = (vi >> 7) << 7
        ru_vec = vu & (_W - 1)
        ri_vec = vi & (_W - 1)
        rpu_vec = pu & (_W - 1)
        rpi_vec = pi & (_W - 1)

        for l in range(_NSLOT):
            @pl.when(g > 0)
            def _(l=l):
                process(rpu_vec, rpi_vec, _L - _NSLOT + l,
                        gp * _L + _L - _NSLOT + l)
            issue(ou_vec, oi_vec, l)

        @pl.when(g == _NGRP // 2)
        def _():
            pltpu.sync_copy(buf, out_hbm.at[pl.ds(base, _HALF)])

        for l in range(_NSLOT, _L):
            process(ru_vec, ri_vec, l - _NSLOT, g * _L + l - _NSLOT)
            issue(ou_vec, oi_vec, l)
        return carry

    lax.fori_loop(0, _NGRP, body, 0)

    g_last = _NGRP - 1
    vu = idx_u[pl.ds(g_last * _L, _L)] & (_W - 1)
    vi = idx_i[pl.ds(g_last * _L, _L)] & (_W - 1)
    for l in range(_NSLOT):
        process(vu, vi, _L - _NSLOT + l, g_last * _L + _L - _NSLOT + l)

    pltpu.sync_copy(buf, out_hbm.at[pl.ds(base + _HALF, _HALF)])


@jax.jit
def kernel(users, items, user_table, item_table):
    mesh = plsc.VectorSubcoreMesh(core_axis_name="c", subcore_axis_name="s")
    run = functools.partial(
        pl.kernel,
        mesh=mesh,
        compiler_params=pltpu.CompilerParams(needs_layout_passes=False),
        out_type=jax.ShapeDtypeStruct((_BATCH, _DIM), jnp.float32),
        scratch_types=[
            pltpu.VMEM((_BPW,), jnp.int32),
            pltpu.VMEM((_BPW,), jnp.int32),
            pltpu.VMEM((_NSLOT, _DIM, _W), jnp.float32),
            pltpu.VMEM((_NSLOT, _DIM, _W), jnp.float32),
            pltpu.VMEM((_HALF, _DIM), jnp.float32),
        ] + [pltpu.SemaphoreType.DMA] * (2 * _NSLOT),
    )(_gmf_body)
    return run(users.astype(jnp.int32), items.astype(jnp.int32),
               user_table.T, item_table.T)
